# baseline (speedup 1.0000x reference)
import jax
import jax.numpy as jnp
from jax.experimental import pallas as pl
from jax.experimental.pallas import tpu as pltpu

IN_F = 4096
OUT_F = 1024
STRIDE = 4
BR = 256


def _tc_body(x0_ref, x1_ref, x2_ref, x3_ref, o_ref):
    n = o_ref.shape[0]
    lane = jax.lax.broadcasted_iota(jnp.int32, (n, 128), 1)
    idx = (lane * STRIDE) % 128
    grp = lane // 32

    del grp
    for g, ref in enumerate((x0_ref, x1_ref, x2_ref, x3_ref)):
        x = ref[:, 0, 0, :]
        y = jnp.take_along_axis(x, idx, axis=1)
        o_ref[:, 32 * g:32 * (g + 1)] = y[:, 32 * g:32 * (g + 1)]


def kernel(input):
    B, S, F = input.shape
    R = B * S
    x = input.reshape(R * 8, STRIDE, 1, 128)
    nrows = R * 8
    brows = BR * 8

    def in_spec(g):
        return pl.BlockSpec((brows, 1, 1, 128), lambda i, g=g: (i, g, 0, 0))

    out = pl.pallas_call(
        _tc_body,
        grid=(R // BR,),
        in_specs=[in_spec(g) for g in range(STRIDE)],
        out_specs=pl.BlockSpec((brows, 128), lambda i: (i, 0)),
        out_shape=jax.ShapeDtypeStruct((nrows, 128), jnp.float32),
    )(x, x, x, x)
    return out.reshape(B, S, OUT_F)


# speedup vs baseline: 8.7277x; 8.7277x over previous
import jax
import jax.numpy as jnp
from jax.experimental import pallas as pl
from jax.experimental.pallas import tpu as pltpu

IN_F = 4096
OUT_F = 1024
STRIDE = 4
BR = 256


def _tc_body(x_ref, o_ref):
    lane = jax.lax.broadcasted_iota(jnp.int32, (BR, 128), 1)
    idx = (lane * STRIDE) % 128
    grp = lane // 32
    for c in range(OUT_F // 128):
        ys = []
        for g in range(STRIDE):
            v = x_ref[:, 512 * c + 128 * g:512 * c + 128 * (g + 1)]
            ys.append(jnp.take_along_axis(v, idx, axis=1))
        y01 = jnp.where(grp == 0, ys[0], ys[1])
        y23 = jnp.where(grp == 2, ys[2], ys[3])
        o_ref[:, 128 * c:128 * (c + 1)] = jnp.where(grp < 2, y01, y23)


def kernel(input):
    B, S, F = input.shape
    R = B * S
    x = input.reshape(R, F)
    out = pl.pallas_call(
        _tc_body,
        grid=(R // BR,),
        in_specs=[pl.BlockSpec((BR, IN_F), lambda i: (i, 0))],
        out_specs=pl.BlockSpec((BR, OUT_F), lambda i: (i, 0)),
        out_shape=jax.ShapeDtypeStruct((R, OUT_F), jnp.float32),
    )(x)
    return out.reshape(B, S, OUT_F)


# speedup vs baseline: 10.1519x; 1.1632x over previous
import jax
import jax.numpy as jnp
from jax.experimental import pallas as pl
from jax.experimental.pallas import tpu as pltpu

IN_F = 4096
OUT_F = 1024
STRIDE = 4
BR = 512


def _tc_body(x_ref, o_ref):
    lane = jax.lax.broadcasted_iota(jnp.int32, (BR, 128), 1)
    idx = (lane * STRIDE) % 128
    grp = lane // 32
    for c in range(OUT_F // 128):
        ys = []
        for g in range(STRIDE):
            v = x_ref[:, 512 * c + 128 * g:512 * c + 128 * (g + 1)]
            ys.append(jnp.take_along_axis(v, idx, axis=1))
        y01 = jnp.where(grp == 0, ys[0], ys[1])
        y23 = jnp.where(grp == 2, ys[2], ys[3])
        o_ref[:, 128 * c:128 * (c + 1)] = jnp.where(grp < 2, y01, y23)


def kernel(input):
    B, S, F = input.shape
    R = B * S
    x = input.reshape(R, F)
    out = pl.pallas_call(
        _tc_body,
        grid=(R // BR,),
        in_specs=[pl.BlockSpec((BR, IN_F), lambda i: (i, 0))],
        out_specs=pl.BlockSpec((BR, OUT_F), lambda i: (i, 0)),
        out_shape=jax.ShapeDtypeStruct((R, OUT_F), jnp.float32),
    )(x)
    return out.reshape(B, S, OUT_F)


# speedup vs baseline: 10.4807x; 1.0324x over previous
import jax
import jax.numpy as jnp
from jax.experimental import pallas as pl
from jax.experimental.pallas import tpu as pltpu

IN_F = 4096
OUT_F = 1024
STRIDE = 4
BR = 1024


def _tc_body(x_ref, o_ref):
    lane = jax.lax.broadcasted_iota(jnp.int32, (BR, 128), 1)
    idx = (lane * STRIDE) % 128
    grp = lane // 32
    for c in range(OUT_F // 128):
        ys = []
        for g in range(STRIDE):
            v = x_ref[:, 512 * c + 128 * g:512 * c + 128 * (g + 1)]
            ys.append(jnp.take_along_axis(v, idx, axis=1))
        y01 = jnp.where(grp == 0, ys[0], ys[1])
        y23 = jnp.where(grp == 2, ys[2], ys[3])
        o_ref[:, 128 * c:128 * (c + 1)] = jnp.where(grp < 2, y01, y23)


def kernel(input):
    B, S, F = input.shape
    R = B * S
    x = input.reshape(R, F)
    out = pl.pallas_call(
        _tc_body,
        grid=(R // BR,),
        in_specs=[pl.BlockSpec((BR, IN_F), lambda i: (i, 0))],
        out_specs=pl.BlockSpec((BR, OUT_F), lambda i: (i, 0)),
        out_shape=jax.ShapeDtypeStruct((R, OUT_F), jnp.float32),
    )(x)
    return out.reshape(B, S, OUT_F)
